# Initial kernel scaffold; baseline (speedup 1.0000x reference)
import jax
import jax.numpy as jnp
from jax.experimental import pallas as pl


def kernel(x):
    raise NotImplementedError("write your pallas kernel here")


# speedup vs baseline: 5.9881x; 5.9881x over previous
import jax
import jax.numpy as jnp
from jax import lax
from jax.experimental import pallas as pl
from jax.experimental.pallas import tpu as pltpu
from jax.experimental.pallas import tpu_sc as plsc
import functools

_PERM = [0, 2, 4, 6, 8, 10, 12, 1, 3, 5, 7, 9, 11, 13]

_N_IMG = 8 * 192
_H = 224
_N_WORKERS = 32
_IMG_PER_W = _N_IMG // _N_WORKERS


def _sc_body(x_hbm, out_hbm, in_buf, out_buf):
    wid = lax.axis_index("s") * 2 + lax.axis_index("c")

    @pl.loop(0, _IMG_PER_W)
    def _per_image(n):
        img = wid * _IMG_PER_W + n
        pltpu.sync_copy(x_hbm.at[img], in_buf)
        for i2 in range(2):
            for j7 in range(7):
                out_r0 = 112 * i2 + 16 * j7
                in_r0 = 32 * j7 + 16 * i2

                @pl.loop(0, 16)
                def _row(t):
                    for wb in range(14):
                        out_buf[out_r0 + t, pl.ds(16 * wb, 16)] = (
                            in_buf[in_r0 + t, pl.ds(16 * _PERM[wb], 16)]
                        )
        pltpu.sync_copy(out_buf, out_hbm.at[img])


def kernel(x):
    x3 = x.reshape(_N_IMG, _H, _H)
    mesh = plsc.VectorSubcoreMesh(core_axis_name="c", subcore_axis_name="s")
    run = pl.kernel(
        _sc_body,
        out_type=jax.ShapeDtypeStruct((_N_IMG, _H, _H), jnp.float32),
        mesh=mesh,
        scratch_types=[
            pltpu.VMEM((_H, _H), jnp.float32),
            pltpu.VMEM((_H, _H), jnp.float32),
        ],
    )
    y = run(x3)
    return y.reshape(x.shape)


# speedup vs baseline: 11.0215x; 1.8405x over previous
import jax
import jax.numpy as jnp
from jax import lax
from jax.experimental import pallas as pl
from jax.experimental.pallas import tpu as pltpu
from jax.experimental.pallas import tpu_sc as plsc

_PERM = [0, 2, 4, 6, 8, 10, 12, 1, 3, 5, 7, 9, 11, 13]

_N_IMG = 8 * 192
_H = 224
_N_WORKERS = 32
_IMG_PER_W = _N_IMG // _N_WORKERS
_TASKS = 2 * _IMG_PER_W


def _sc_body(x_hbm, out_hbm, ib0, ib1, ob0, ob1, rs0, rs1, ws0, ws1):
    wid = lax.axis_index("s") * 2 + lax.axis_index("c")
    img0 = wid * _IMG_PER_W

    def start_read(t, ibuf, rsem):
        img = img0 + t // 2
        i2 = t % 2
        for j7 in range(7):
            pltpu.async_copy(
                x_hbm.at[img, pl.ds(32 * j7 + 16 * i2, 16)],
                ibuf.at[pl.ds(16 * j7, 16)],
                rsem,
            )

    def wait_read(ibuf, rsem):
        pltpu.make_async_copy(x_hbm.at[img0, pl.ds(0, 112)], ibuf, rsem).wait()

    def shuffle(ibuf, obuf):
        @pl.loop(0, 112, unroll=4)
        def _row(r):
            for wb in range(14):
                obuf[r, pl.ds(16 * wb, 16)] = ibuf[r, pl.ds(16 * _PERM[wb], 16)]

    def start_write(t, obuf, wsem):
        img = img0 + t // 2
        i2 = t % 2
        pltpu.async_copy(obuf, out_hbm.at[img, pl.ds(112 * i2, 112)], wsem)

    def wait_write(obuf, wsem):
        pltpu.make_async_copy(obuf, out_hbm.at[img0, pl.ds(0, 112)], wsem).wait()

    start_read(0, ib0, rs0)

    @pl.loop(0, _TASKS, step=2)
    def _pipe(g):
        start_read(g + 1, ib1, rs1)
        wait_read(ib0, rs0)

        @pl.when(g > 0)
        def _():
            wait_write(ob0, ws0)

        shuffle(ib0, ob0)
        start_write(g, ob0, ws0)

        @pl.when(g + 2 < _TASKS)
        def _():
            start_read(g + 2, ib0, rs0)

        wait_read(ib1, rs1)

        @pl.when(g > 0)
        def _():
            wait_write(ob1, ws1)

        shuffle(ib1, ob1)
        start_write(g + 1, ob1, ws1)

    wait_write(ob0, ws0)
    wait_write(ob1, ws1)


def kernel(x):
    x3 = x.reshape(_N_IMG, _H, _H)
    mesh = plsc.VectorSubcoreMesh(core_axis_name="c", subcore_axis_name="s")
    run = pl.kernel(
        _sc_body,
        out_type=jax.ShapeDtypeStruct((_N_IMG, _H, _H), jnp.float32),
        mesh=mesh,
        scratch_types=[
            pltpu.VMEM((112, _H), jnp.float32),
            pltpu.VMEM((112, _H), jnp.float32),
            pltpu.VMEM((112, _H), jnp.float32),
            pltpu.VMEM((112, _H), jnp.float32),
            pltpu.SemaphoreType.DMA,
            pltpu.SemaphoreType.DMA,
            pltpu.SemaphoreType.DMA,
            pltpu.SemaphoreType.DMA,
        ],
    )
    y = run(x3)
    return y.reshape(x.shape)
